# Initial kernel scaffold; baseline (speedup 1.0000x reference)
import jax
import jax.numpy as jnp
from jax.experimental import pallas as pl


def kernel(adj, node_feats, eigvec, W_lap, graph_tok):
    raise NotImplementedError("write your pallas kernel here")


# speedup vs baseline: 1.1451x; 1.1451x over previous
import jax
import jax.numpy as jnp
from jax.experimental import pallas as pl

B, N, D_MODEL, LAP_DIM = 8, 2048, 128, 8


def _seq_kernel(nf_ref, ev_ref, w_ref, g_ref, out_ref):
    ev = ev_ref[0]
    w = w_ref[...]
    lap = jax.lax.dot_general(
        ev, w, (((1,), (1,)), ((), ())),
        preferred_element_type=jnp.float32)
    out_ref[0, 0, :] = g_ref[0, 0, :]
    out_ref[0, pl.ds(1, N), :] = nf_ref[0] + lap


def kernel(adj, node_feats, eigvec, W_lap, graph_tok):
    b, n, _ = adj.shape
    d = node_feats.shape[-1]
    seq = pl.pallas_call(
        _seq_kernel,
        grid=(b,),
        in_specs=[
            pl.BlockSpec((1, n, d), lambda i: (i, 0, 0)),
            pl.BlockSpec((1, n, eigvec.shape[-1]), lambda i: (i, 0, 0)),
            pl.BlockSpec(W_lap.shape, lambda i: (0, 0)),
            pl.BlockSpec(graph_tok.shape, lambda i: (0, 0, 0)),
        ],
        out_specs=pl.BlockSpec((1, 1 + n, d), lambda i: (i, 0, 0)),
        out_shape=jax.ShapeDtypeStruct((b, 1 + n, d), jnp.float32),
    )(node_feats, eigvec, W_lap, graph_tok)
    pad_mask = jnp.zeros((b, 1 + n), dtype=bool)
    return seq, pad_mask


# speedup vs baseline: 1.2171x; 1.0629x over previous
import jax
import jax.numpy as jnp
from jax.experimental import pallas as pl
from jax.experimental.pallas import tpu as pltpu

B, N, D_MODEL, LAP_DIM = 8, 2048, 128, 8


def _fused_kernel(nf_hbm, ev_ref, w_ref, g_ref, out_hbm,
                  nf_v, out_v, in_sems, out_sems, gout_sems):
    for b in range(B):
        pltpu.make_async_copy(nf_hbm.at[b], nf_v.at[b], in_sems.at[b]).start()
    for b in range(B):
        pltpu.make_async_copy(
            g_ref.at[0], out_hbm.at[b, pl.ds(0, 1), :], gout_sems.at[b]
        ).start()
    w = w_ref[...]
    for b in range(B):
        pltpu.make_async_copy(nf_hbm.at[b], nf_v.at[b], in_sems.at[b]).wait()
        lap = jax.lax.dot_general(
            ev_ref[b], w, (((1,), (1,)), ((), ())),
            preferred_element_type=jnp.float32)
        out_v[b] = nf_v[b] + lap
        pltpu.make_async_copy(
            out_v.at[b], out_hbm.at[b, pl.ds(1, N), :], out_sems.at[b]
        ).start()
    for b in range(B):
        pltpu.make_async_copy(
            out_v.at[b], out_hbm.at[b, pl.ds(1, N), :], out_sems.at[b]
        ).wait()
        pltpu.make_async_copy(
            g_ref.at[0], out_hbm.at[b, pl.ds(0, 1), :], gout_sems.at[b]
        ).wait()


def kernel(adj, node_feats, eigvec, W_lap, graph_tok):
    b, n, _ = adj.shape
    d = node_feats.shape[-1]
    lap_dim = eigvec.shape[-1]
    seq = pl.pallas_call(
        _fused_kernel,
        in_specs=[
            pl.BlockSpec(memory_space=pl.ANY),
            pl.BlockSpec((b, n, lap_dim), lambda: (0, 0, 0)),
            pl.BlockSpec(W_lap.shape, lambda: (0, 0)),
            pl.BlockSpec(graph_tok.shape, lambda: (0, 0, 0)),
        ],
        out_specs=pl.BlockSpec(memory_space=pl.ANY),
        out_shape=jax.ShapeDtypeStruct((b, 1 + n, d), jnp.float32),
        scratch_shapes=[
            pltpu.MemorySpace.VMEM((b, n, d), jnp.float32),
            pltpu.MemorySpace.VMEM((b, n, d), jnp.float32),
            pltpu.SemaphoreType.DMA((b,)),
            pltpu.SemaphoreType.DMA((b,)),
            pltpu.SemaphoreType.DMA((b,)),
        ],
    )(node_feats, eigvec, W_lap, graph_tok)
    pad_mask = jnp.zeros((b, 1 + n), dtype=bool)
    return seq, pad_mask
